# baseline (speedup 1.0000x reference)
import functools
import math

import jax
import jax.numpy as jnp
import numpy as np
from jax import lax
from jax.experimental import pallas as pl
from jax.experimental.pallas import tpu as pltpu
from jax.experimental.pallas import tpu_sc as plsc

_VOCAB = 100000
_D = 768
_B = 4
_S = 4096
_NC = 2
_NS = 16
_NW = _NC * _NS
_POS_PER_W = _S // _NW
_P = 8
_NUNITS = _POS_PER_W // _P
_ROWS = _B * _P
_LANES = 16
_NCHK = _D // (2 * _LANES)
_SCALE = math.sqrt(_D)


def _make_pe_np(max_len, d_model):
    pe = np.zeros((max_len, d_model), dtype=np.float32)
    position = np.arange(0, max_len, dtype=np.float32)[:, None]
    div_term = np.exp(
        np.arange(0, d_model, 2, dtype=np.float32) * -(math.log(10000.0) / d_model)
    )
    pe[:, 0::2] = np.sin(position * div_term)
    pe[:, 1::2] = np.cos(position * div_term)
    return pe


def _pack_pe_np():
    pe = _make_pe_np(_S, _D)
    bf = pe.astype(jnp.bfloat16)
    u16 = bf.view(np.uint16).reshape(_S, _NCHK, 2, _LANES)
    lo = u16[:, :, 0, :].astype(np.uint32)
    hi = u16[:, :, 1, :].astype(np.uint32)
    packed = (lo | (hi << np.uint32(16))).reshape(_S, _D // 2)
    return packed.view(np.int32)


_PE_PACKED = _pack_pe_np()


def _body(x_hbm, table_hbm, pe_hbm, out_hbm, idx_v, pe_v, gbuf, obuf,
          gsem, ssem, psem):
    wid = lax.axis_index("s") * _NC + lax.axis_index("c")
    pos0 = wid * _POS_PER_W

    for b in range(_B):
        pltpu.sync_copy(x_hbm.at[b, wid], idx_v.at[b])

    def gather(u):
        return [
            pltpu.async_copy(
                table_hbm.at[idx_v.at[b, u]],
                gbuf.at[u % 2, pl.ds(b * _P, _P)],
                gsem,
            )
            for b in range(_B)
        ]

    def pe_load(u):
        return pltpu.async_copy(
            pe_hbm.at[pl.ds(pos0 + u * _P, _P)], pe_v.at[u % 2], psem)

    def scatter1(u, b):
        return pltpu.async_copy(
            obuf.at[u % 2, pl.ds(b * _P, _P)],
            out_hbm.at[pl.ds(b * _S + pos0 + u * _P, _P)],
            ssem,
        )

    g = {0: gather(0), 1: gather(1)}
    p = {0: pe_load(0), 1: pe_load(1)}
    s = {}
    for u in range(_NUNITS):
        slot = u % 2
        gslot = u % 2
        for cp in g[u]:
            cp.wait()
        p[u].wait()
        for b in range(_B):
            if u >= 2:
                s[(u - 2, b)].wait()

            @plsc.parallel_loop(0, _NCHK, 1)
            def c_body(c, _slot=slot, _g=gslot, _b=b):
                sl_pe = pl.ds(c * _LANES, _LANES)
                sl_a = pl.ds(c * (2 * _LANES), _LANES)
                sl_b = pl.ds(c * (2 * _LANES) + _LANES, _LANES)
                ws = [pe_v[_slot, i, sl_pe] for i in range(_P)]
                ras = [gbuf[_g, _b * _P + i, sl_a] for i in range(_P)]
                rbs = [gbuf[_g, _b * _P + i, sl_b] for i in range(_P)]
                pas = [
                    lax.bitcast_convert_type(lax.shift_left(w, 16), jnp.float32)
                    for w in ws
                ]
                pbs = [
                    lax.bitcast_convert_type(
                        lax.bitwise_and(w, jnp.int32(-65536)), jnp.float32)
                    for w in ws
                ]
                for i in range(_P):
                    obuf[_slot, _b * _P + i, sl_a] = ras[i] * _SCALE + pas[i]
                    obuf[_slot, _b * _P + i, sl_b] = rbs[i] * _SCALE + pbs[i]

            s[(u, b)] = scatter1(u, b)
        if u + 2 < _NUNITS:
            g[u + 2] = gather(u + 2)
            p[u + 2] = pe_load(u + 2)
    for u in (_NUNITS - 2, _NUNITS - 1):
        for b in range(_B):
            s[(u, b)].wait()


def _build(interpret=False):
    mesh = plsc.VectorSubcoreMesh(core_axis_name="c", subcore_axis_name="s")
    return pl.kernel(
        _body,
        out_type=jax.ShapeDtypeStruct((_B * _S, _D), jnp.float32),
        mesh=mesh,
        scratch_types=[
            pltpu.VMEM((_B, _NUNITS, _P), jnp.int32),
            pltpu.VMEM((2, _P, _D // 2), jnp.int32),
            pltpu.VMEM((2, _ROWS, _D), jnp.float32),
            pltpu.VMEM((2, _ROWS, _D), jnp.float32),
            pltpu.SemaphoreType.DMA,
            pltpu.SemaphoreType.DMA,
            pltpu.SemaphoreType.DMA,
        ],
        interpret=interpret,
    )


_sc_embed = _build()


def kernel(x, table):
    x2 = x.astype(jnp.int32).reshape(_B, _NW, _NUNITS, _P)
    out = _sc_embed(x2, table, jnp.asarray(_PE_PACKED))
    return out.reshape(_B, _S, _D)


# speedup vs baseline: 1.0506x; 1.0506x over previous
import functools
import math

import jax
import jax.numpy as jnp
import numpy as np
from jax import lax
from jax.experimental import pallas as pl
from jax.experimental.pallas import tpu as pltpu
from jax.experimental.pallas import tpu_sc as plsc

_VOCAB = 100000
_D = 768
_B = 4
_S = 4096
_NC = 2
_NS = 16
_NW = _NC * _NS
_POS_PER_W = _S // _NW
_P = 8
_NUNITS = _POS_PER_W // _P
_ROWS = _B * _P
_LANES = 16
_NCHK = _D // (2 * _LANES)
_SCALE = math.sqrt(_D)


def _make_pe_np(max_len, d_model):
    pe = np.zeros((max_len, d_model), dtype=np.float32)
    position = np.arange(0, max_len, dtype=np.float32)[:, None]
    div_term = np.exp(
        np.arange(0, d_model, 2, dtype=np.float32) * -(math.log(10000.0) / d_model)
    )
    pe[:, 0::2] = np.sin(position * div_term)
    pe[:, 1::2] = np.cos(position * div_term)
    return pe


def _pack_pe_np():
    pe = _make_pe_np(_S, _D)
    bf = pe.astype(jnp.bfloat16)
    u16 = bf.view(np.uint16).reshape(_S, _NCHK, 2, _LANES)
    lo = u16[:, :, 0, :].astype(np.uint32)
    hi = u16[:, :, 1, :].astype(np.uint32)
    packed = (lo | (hi << np.uint32(16))).reshape(_S, _D // 2)
    return packed.view(np.int32)


_PE_PACKED = _pack_pe_np()


def _body(x_hbm, table_hbm, pe_hbm, out_hbm, idx_v, pe_v, gbuf, obuf,
          gsem, ssem, psem):
    wid = lax.axis_index("s") * _NC + lax.axis_index("c")
    pos0 = wid * _POS_PER_W

    pltpu.sync_copy(x_hbm.at[wid], idx_v)

    def gather(u):
        return pltpu.async_copy(table_hbm.at[idx_v.at[u]], gbuf.at[u % 3], gsem)

    def pe_load(u):
        return pltpu.async_copy(
            pe_hbm.at[pl.ds(pos0 + u * _P, _P)], pe_v.at[u % 2], psem)

    def scatter1(u, b):
        return pltpu.async_copy(
            obuf.at[u % 2, pl.ds(b * _P, _P)],
            out_hbm.at[pl.ds(b * _S + pos0 + u * _P, _P)],
            ssem,
        )

    g = {0: gather(0), 1: gather(1), 2: gather(2)}
    p = {0: pe_load(0), 1: pe_load(1)}
    s = {}
    for u in range(_NUNITS):
        slot = u % 2
        gslot = u % 3
        g[u].wait()
        p[u].wait()
        for b in range(_B):
            if u >= 2:
                s[(u - 2, b)].wait()

            @plsc.parallel_loop(0, _NCHK, 1)
            def c_body(c, _slot=slot, _g=gslot, _b=b):
                sl_pe = pl.ds(c * _LANES, _LANES)
                sl_a = pl.ds(c * (2 * _LANES), _LANES)
                sl_b = pl.ds(c * (2 * _LANES) + _LANES, _LANES)
                ws = [pe_v[_slot, i, sl_pe] for i in range(_P)]
                ras = [gbuf[_g, _b * _P + i, sl_a] for i in range(_P)]
                rbs = [gbuf[_g, _b * _P + i, sl_b] for i in range(_P)]
                pas = [
                    lax.bitcast_convert_type(lax.shift_left(w, 16), jnp.float32)
                    for w in ws
                ]
                pbs = [
                    lax.bitcast_convert_type(
                        lax.bitwise_and(w, jnp.int32(-65536)), jnp.float32)
                    for w in ws
                ]
                for i in range(_P):
                    obuf[_slot, _b * _P + i, sl_a] = ras[i] * _SCALE + pas[i]
                    obuf[_slot, _b * _P + i, sl_b] = rbs[i] * _SCALE + pbs[i]

            s[(u, b)] = scatter1(u, b)
        if u + 3 < _NUNITS:
            g[u + 3] = gather(u + 3)
        if u + 2 < _NUNITS:
            p[u + 2] = pe_load(u + 2)
    for u in (_NUNITS - 2, _NUNITS - 1):
        for b in range(_B):
            s[(u, b)].wait()


def _build(interpret=False):
    mesh = plsc.VectorSubcoreMesh(core_axis_name="c", subcore_axis_name="s")
    return pl.kernel(
        _body,
        out_type=jax.ShapeDtypeStruct((_B * _S, _D), jnp.float32),
        mesh=mesh,
        scratch_types=[
            pltpu.VMEM((_NUNITS, _ROWS), jnp.int32),
            pltpu.VMEM((2, _P, _D // 2), jnp.int32),
            pltpu.VMEM((3, _ROWS, _D), jnp.float32),
            pltpu.VMEM((2, _ROWS, _D), jnp.float32),
            pltpu.SemaphoreType.DMA,
            pltpu.SemaphoreType.DMA,
            pltpu.SemaphoreType.DMA,
        ],
        interpret=interpret,
    )


_sc_embed = _build()


def kernel(x, table):
    x2 = (
        x.astype(jnp.int32)
        .reshape(_B, _NW, _NUNITS, _P)
        .transpose(1, 2, 0, 3)
        .reshape(_NW, _NUNITS, _ROWS)
    )
    out = _sc_embed(x2, table, jnp.asarray(_PE_PACKED))
    return out.reshape(_B, _S, _D)
